# baseline (speedup 1.0000x reference)
import jax
import jax.numpy as jnp
from jax.experimental import pallas as pl

_G = 2


def kernel(x_fea, graph_pool_avg, padded_nei, adj, candidate, h_g_m_pooled,
           mask_operation,
           gin0_w1, gin0_b1, gin0_w2, gin0_b2,
           gin1_w1, gin1_b1, gin1_w2, gin1_b2,
           gin2_w1, gin2_b1, gin2_w2, gin2_b2,
           a_w1, a_b1, a_w2, a_b2, a_w3, a_b3,
           c_w1, c_b1, c_w2, c_b2, c_w3, c_b3):
    B, T, _ = adj.shape
    NJ = candidate.shape[1]
    DIN = x_fea.shape[1]
    H = gin0_w2.shape[0]
    G = _G
    f32 = jnp.float32

    cand = candidate.astype(jnp.int32)
    mask_f = mask_operation.astype(f32)

    def lanes(w):
        w = w.reshape(1, -1) if w.ndim == 1 else w
        return jnp.pad(w, ((0, 0), (0, H - w.shape[1]))) if w.shape[1] < H else w

    order = (('g0w1', gin0_w1), ('g0b1', gin0_b1), ('g0w2', gin0_w2),
             ('g0b2', gin0_b2),
             ('g1w1', gin1_w1), ('g1b1', gin1_b1), ('g1w2', gin1_w2),
             ('g1b2', gin1_b2),
             ('g2w1', gin2_w1), ('g2b1', gin2_b1), ('g2w2', gin2_w2),
             ('g2b2', gin2_b2),
             ('aw1', a_w1), ('ab1', a_b1), ('aw2', a_w2), ('ab2', a_b2),
             ('aw3', a_w3), ('ab3', a_b3),
             ('cw1', c_w1), ('cb1', c_b1), ('cw2', c_w2), ('cb2', c_b2),
             ('cw3', c_w3), ('cb3', c_b3))
    offs, pieces, pos = {}, [], 0
    for name, w in order:
        w = lanes(w)
        rows = w.shape[0]
        arows = -(-rows // 8) * 8
        offs[name] = (pos, arows, rows)
        pieces.append(jnp.pad(w, ((0, arows - rows), (0, 0))))
        pos += arows
    wpack = jnp.concatenate(pieces, axis=0)

    def fused_step(x_ref, adj_ref, cand_ref, mask_ref, hgm_ref, w_ref,
                   ti_ref, ai_ref, la_ref, pr_ref, hg_ref, jv_ref):
        def seg(name):
            o, arows, rows = offs[name]
            v = w_ref[o:o + arows, :]
            return v[:rows] if rows != arows else v

        step = pl.program_id(0)
        rows_ = [step * G + g for g in range(G)]
        R = range(G)
        dot = lambda p, q: jnp.dot(p, q, preferred_element_type=f32)
        gin = tuple((seg(f'g{l}w1'), seg(f'g{l}b1'),
                     seg(f'g{l}w2'), seg(f'g{l}b2')) for l in range(3))

        A = [adj_ref[g] for g in R]
        h = [x_ref[pl.ds(g * T, T), :] for g in R]

        for li, (w1, b1, w2, b2) in enumerate(gin):
            w1 = w1[:DIN] if li == 0 else w1
            neigh = [dot(A[g], h[g]) for g in R]
            pooled = [neigh[g] + h[g] for g in R]
            m = [jnp.maximum(dot(pooled[g], w1) + b1, 0.0) for g in R]
            m = [dot(m[g], w2) + b2 for g in R]
            h = [jnp.maximum(m[g], 0.0) for g in R]

        pool = jnp.full((1, T), 1.0 / T, f32)
        hg = [dot(pool, h[g]) for g in R]

        cand_row = [cand_ref[pl.ds(rows_[g], 1), :] for g in R]
        cand_col = [jnp.transpose(cand_row[g]) for g in R]
        iota_t = jax.lax.broadcasted_iota(jnp.int32, (NJ, T), 1)
        onehot = [(iota_t == cand_col[g]).astype(f32) for g in R]
        cf = [dot(onehot[g], h[g]) for g in R]

        aw1 = seg('aw1')
        aw1a, aw1b, aw1c = aw1[:H], aw1[H:2 * H], aw1[2 * H:]
        ab1 = seg('ab1')
        aw2, ab2 = seg('aw2'), seg('ab2')
        aw3, ab3 = seg('aw3')[:, :1], seg('ab3')[:, :1]
        hgm = [hgm_ref[pl.ds(rows_[g], 1), :] for g in R]
        row = [dot(hg[g], aw1b) + dot(hgm[g], aw1c) + ab1 for g in R]
        t1 = [jnp.tanh(dot(cf[g], aw1a) + row[g]) for g in R]
        t2 = [jnp.tanh(dot(t1[g], aw2) + ab2) for g in R]
        sc = [dot(t2[g], aw3) + ab3 for g in R]
        scr = [jnp.transpose(sc[g]) - mask_ref[pl.ds(rows_[g], 1), :] * 1e30
               for g in R]

        smax = [jnp.max(scr[g], axis=1, keepdims=True) for g in R]
        e = [jnp.exp(scr[g] - smax[g]) for g in R]
        esum = [jnp.sum(e[g], axis=1, keepdims=True) for g in R]
        prob = [e[g] / esum[g] for g in R]

        iota_nj = jax.lax.broadcasted_iota(jnp.int32, (1, NJ), 1)
        pmax = [jnp.max(prob[g], axis=1, keepdims=True) for g in R]
        am = [jnp.min(jnp.where(prob[g] == pmax[g], iota_nj, NJ),
                      axis=1, keepdims=True) for g in R]
        task = [jnp.sum(jnp.where(iota_nj == am[g], cand_row[g], 0),
                        axis=1, keepdims=True) for g in R]
        la = [jnp.log(pmax[g] + 1e-10) for g in R]

        cw1, cb1 = seg('cw1'), seg('cb1')
        cw2, cb2 = seg('cw2'), seg('cb2')
        cw3, cb3 = seg('cw3')[:, :4], seg('cb3')[:, :4]
        v1 = [jnp.tanh(dot(hg[g], cw1) + cb1) for g in R]
        v2 = [jnp.tanh(dot(v1[g], cw2) + cb2) for g in R]
        jv = [dot(v2[g], cw3) + cb3 for g in R]

        for g in R:
            ti_ref[g] = task[g]
            ai_ref[g] = am[g]
            la_ref[g] = la[g]
            pr_ref[g] = prob[g]
            hg_ref[g] = hg[g]
            jv_ref[g] = jv[g]

    def full(w):
        nd = w.ndim
        return pl.BlockSpec(w.shape, lambda b, _n=nd: (0,) * _n)

    in_specs = [
        pl.BlockSpec((G * T, DIN), lambda b: (b, 0)),
        pl.BlockSpec((G, T, T), lambda b: (b, 0, 0)),
        full(cand),
        full(mask_f),
        full(h_g_m_pooled),
        full(wpack),
    ]

    out_shapes = (
        jax.ShapeDtypeStruct((B, 1, 1), jnp.int32),
        jax.ShapeDtypeStruct((B, 1, 1), jnp.int32),
        jax.ShapeDtypeStruct((B, 1, 1), f32),
        jax.ShapeDtypeStruct((B, 1, NJ), f32),
        jax.ShapeDtypeStruct((B, 1, H), f32),
        jax.ShapeDtypeStruct((B, 1, 4), f32),
    )
    out_specs = (
        pl.BlockSpec((G, 1, 1), lambda b: (b, 0, 0)),
        pl.BlockSpec((G, 1, 1), lambda b: (b, 0, 0)),
        pl.BlockSpec((G, 1, 1), lambda b: (b, 0, 0)),
        pl.BlockSpec((G, 1, NJ), lambda b: (b, 0, 0)),
        pl.BlockSpec((G, 1, H), lambda b: (b, 0, 0)),
        pl.BlockSpec((G, 1, 4), lambda b: (b, 0, 0)),
    )

    ti, ai, la, pr, hg, jv = pl.pallas_call(
        fused_step,
        grid=(B // G,),
        in_specs=in_specs,
        out_specs=out_specs,
        out_shape=out_shapes,
    )(x_fea, adj, cand, mask_f, h_g_m_pooled, wpack)

    return (ti.reshape(B), ai.reshape(B), la.reshape(B),
            pr.reshape(B, NJ), hg.reshape(B, H), jv.reshape(B, 4))


# speedup vs baseline: 1.1694x; 1.1694x over previous
import jax
import jax.numpy as jnp
from jax.experimental import pallas as pl

_G = 2


def _fused_step(
    x_ref, adj_ref, cand_ref, mask_ref, hgm_ref,
    g0w1_ref, g0b1_ref, g0w2_ref, g0b2_ref,
    g1w1_ref, g1b1_ref, g1w2_ref, g1b2_ref,
    g2w1_ref, g2b1_ref, g2w2_ref, g2b2_ref,
    aw1_ref, ab1_ref, aw2_ref, ab2_ref, aw3_ref, ab3_ref,
    cw1_ref, cb1_ref, cw2_ref, cb2_ref, cw3_ref, cb3_ref,
    ti_ref, ai_ref, la_ref, pr_ref, hg_ref, jv_ref,
):
    f32 = jnp.float32
    G, T, _ = adj_ref.shape
    H = g0w2_ref.shape[0]
    nj = cand_ref.shape[1]
    step = pl.program_id(0)
    rows = [step * G + g for g in range(G)]
    R = range(G)
    dot = lambda p, q: jnp.dot(p, q, preferred_element_type=f32)
    gin = ((g0w1_ref, g0b1_ref, g0w2_ref, g0b2_ref),
           (g1w1_ref, g1b1_ref, g1w2_ref, g1b2_ref),
           (g2w1_ref, g2b1_ref, g2w2_ref, g2b2_ref))

    A = [adj_ref[g] for g in R]
    h = [x_ref[pl.ds(g * T, T), :] for g in R]

    for w1_ref, b1_ref, w2_ref, b2_ref in gin:
        w1, b1, w2, b2 = w1_ref[...], b1_ref[...], w2_ref[...], b2_ref[...]
        neigh = [dot(A[g], h[g]) for g in R]
        pooled = [neigh[g] + h[g] for g in R]
        m = [jnp.maximum(dot(pooled[g], w1) + b1, 0.0) for g in R]
        m = [dot(m[g], w2) + b2 for g in R]
        h = [jnp.maximum(m[g], 0.0) for g in R]

    pool = jnp.full((1, T), 1.0 / T, f32)
    hg = [dot(pool, h[g]) for g in R]

    cand_row = [cand_ref[pl.ds(rows[g], 1), :] for g in R]
    cand_col = [jnp.transpose(cand_row[g]) for g in R]
    iota_t = jax.lax.broadcasted_iota(jnp.int32, (nj, T), 1)
    onehot = [(iota_t == cand_col[g]).astype(f32) for g in R]
    cf = [dot(onehot[g], h[g]) for g in R]

    aw1a = aw1_ref[0:H, :]
    aw1b = aw1_ref[H:2 * H, :]
    aw1c = aw1_ref[2 * H:3 * H, :]
    hgm = [hgm_ref[pl.ds(rows[g], 1), :] for g in R]
    row = [dot(hg[g], aw1b) + dot(hgm[g], aw1c) + ab1_ref[...] for g in R]
    t1 = [jnp.tanh(dot(cf[g], aw1a) + row[g]) for g in R]
    t2 = [jnp.tanh(dot(t1[g], aw2_ref[...]) + ab2_ref[...]) for g in R]
    sc = [dot(t2[g], aw3_ref[...]) + ab3_ref[...] for g in R]
    scr = [jnp.transpose(sc[g]) - mask_ref[pl.ds(rows[g], 1), :] * 1e30
           for g in R]

    smax = [jnp.max(scr[g], axis=1, keepdims=True) for g in R]
    e = [jnp.exp(scr[g] - smax[g]) for g in R]
    esum = [jnp.sum(e[g], axis=1, keepdims=True) for g in R]
    prob = [e[g] / esum[g] for g in R]

    iota_nj = jax.lax.broadcasted_iota(jnp.int32, (1, nj), 1)
    pmax = [jnp.max(prob[g], axis=1, keepdims=True) for g in R]
    am = [jnp.min(jnp.where(prob[g] == pmax[g], iota_nj, nj),
                  axis=1, keepdims=True) for g in R]
    task = [jnp.sum(jnp.where(iota_nj == am[g], cand_row[g], 0),
                    axis=1, keepdims=True) for g in R]
    la = [jnp.log(pmax[g] + 1e-10) for g in R]

    v1 = [jnp.tanh(dot(hg[g], cw1_ref[...]) + cb1_ref[...]) for g in R]
    v2 = [jnp.tanh(dot(v1[g], cw2_ref[...]) + cb2_ref[...]) for g in R]
    jv = [dot(v2[g], cw3_ref[...]) + cb3_ref[...] for g in R]

    for g in R:
        ti_ref[g] = task[g]
        ai_ref[g] = am[g]
        la_ref[g] = la[g]
        pr_ref[g] = prob[g]
        hg_ref[g] = hg[g]
        jv_ref[g] = jv[g]


def kernel(x_fea, graph_pool_avg, padded_nei, adj, candidate, h_g_m_pooled,
           mask_operation,
           gin0_w1, gin0_b1, gin0_w2, gin0_b2,
           gin1_w1, gin1_b1, gin1_w2, gin1_b2,
           gin2_w1, gin2_b1, gin2_w2, gin2_b2,
           a_w1, a_b1, a_w2, a_b2, a_w3, a_b3,
           c_w1, c_b1, c_w2, c_b2, c_w3, c_b3):
    B, T, _ = adj.shape
    NJ = candidate.shape[1]
    DIN = x_fea.shape[1]
    H = gin0_w2.shape[0]
    G = _G
    f32 = jnp.float32

    cand = candidate.astype(jnp.int32)
    mask_f = mask_operation.astype(f32)
    r2 = lambda v: v.reshape(1, -1)

    def full(w):
        nd = w.ndim
        return pl.BlockSpec(w.shape, lambda b, _n=nd: (0,) * _n)

    weights = (gin0_w1, r2(gin0_b1), gin0_w2, r2(gin0_b2),
               gin1_w1, r2(gin1_b1), gin1_w2, r2(gin1_b2),
               gin2_w1, r2(gin2_b1), gin2_w2, r2(gin2_b2),
               a_w1, r2(a_b1), a_w2, r2(a_b2), a_w3, r2(a_b3),
               c_w1, r2(c_b1), c_w2, r2(c_b2), c_w3, r2(c_b3))

    in_specs = [
        pl.BlockSpec((G * T, DIN), lambda b: (b, 0)),
        pl.BlockSpec((G, T, T), lambda b: (b, 0, 0)),
        full(cand),
        full(mask_f),
        full(h_g_m_pooled),
    ] + [full(w) for w in weights]

    out_shapes = (
        jax.ShapeDtypeStruct((B, 1, 1), jnp.int32),
        jax.ShapeDtypeStruct((B, 1, 1), jnp.int32),
        jax.ShapeDtypeStruct((B, 1, 1), f32),
        jax.ShapeDtypeStruct((B, 1, NJ), f32),
        jax.ShapeDtypeStruct((B, 1, H), f32),
        jax.ShapeDtypeStruct((B, 1, 4), f32),
    )
    out_specs = (
        pl.BlockSpec((G, 1, 1), lambda b: (b, 0, 0)),
        pl.BlockSpec((G, 1, 1), lambda b: (b, 0, 0)),
        pl.BlockSpec((G, 1, 1), lambda b: (b, 0, 0)),
        pl.BlockSpec((G, 1, NJ), lambda b: (b, 0, 0)),
        pl.BlockSpec((G, 1, H), lambda b: (b, 0, 0)),
        pl.BlockSpec((G, 1, 4), lambda b: (b, 0, 0)),
    )

    ti, ai, la, pr, hg, jv = pl.pallas_call(
        _fused_step,
        grid=(B // G,),
        in_specs=in_specs,
        out_specs=out_specs,
        out_shape=out_shapes,
    )(x_fea, adj, cand, mask_f, h_g_m_pooled, *weights)

    return (ti.reshape(B), ai.reshape(B), la.reshape(B),
            pr.reshape(B, NJ), hg.reshape(B, H), jv.reshape(B, 4))
